# baseline (speedup 1.0000x reference)
import jax
import jax.numpy as jnp
from jax import lax
from jax.experimental import pallas as pl
from jax.experimental.pallas import tpu as pltpu
from jax.experimental.pallas import tpu_sc as plsc

_NUM_FIELDS = 26
_FIELD_DIM = 100000
_D = 16
_B = 16384
_NC = 2
_NS = 16
_NW = _NC * _NS
_BPW = _B // _NW
_CHUNK = 64
_NCHUNK = _BPW // _CHUNK
_RPC = _CHUNK * _NUM_FIELDS
_IDX_BLK = 128
_NBLK = _RPC // _IDX_BLK
_GRP = 16 * _NUM_FIELDS


def _fm_body(embed_hbm, lin_hbm, idx_hbm, bias_hbm, out_hbm,
             idx_a, idx_b, rows_a, rows_b, linv_a, linv_b,
             bias_v, outb,
             sem_ea, sem_eb, sem_la, sem_lb):
    wid = lax.axis_index("s") * _NC + lax.axis_index("c")
    base = wid * _BPW
    idx_base = wid * (_BPW * _NUM_FIELDS)

    pltpu.sync_copy(bias_hbm, bias_v)
    bias16 = bias_v[...]
    iota16 = lax.iota(jnp.int32, 16)

    idx_bufs = (idx_a, idx_b)
    row_bufs = (rows_a, rows_b)
    lin_bufs = (linv_a, linv_b)
    sem_e = (sem_ea, sem_eb)
    sem_l = (sem_la, sem_lb)

    def fire(gc, b):
        pltpu.sync_copy(idx_hbm.at[pl.ds(idx_base + gc * _RPC, _RPC)],
                        idx_bufs[b])
        for j in range(_NBLK):
            blk = idx_bufs[b].at[pl.ds(j * _IDX_BLK, _IDX_BLK)]
            pltpu.async_copy(embed_hbm.at[blk],
                             row_bufs[b].at[pl.ds(j * _IDX_BLK, _IDX_BLK)],
                             sem_e[b])
            pltpu.async_copy(lin_hbm.at[blk],
                             lin_bufs[b].at[pl.ds(j * _IDX_BLK, _IDX_BLK)],
                             sem_l[b])

    def drain(b):
        pltpu.make_async_copy(embed_hbm.at[pl.ds(0, _RPC)], row_bufs[b],
                              sem_e[b]).wait()
        pltpu.make_async_copy(lin_hbm.at[pl.ds(0, _RPC)], lin_bufs[b],
                              sem_l[b]).wait()

    def compute(gc, b):
        rows = row_bufs[b]
        linv = lin_bufs[b]

        def group(t, _):
            grp = t * _GRP
            fmacc = jnp.zeros((16,), jnp.float32)
            for e in range(16):
                r = rows[grp + e]
                s = r
                q = r * r
                for f in range(1, _NUM_FIELDS):
                    r = rows[grp + f * 16 + e]
                    s = s + r
                    q = q + r * r
                v = s * s - q
                for sh in (8, 4, 2, 1):
                    v = v + v[jnp.bitwise_xor(iota16, sh)]
                fmacc = jnp.where(iota16 == e, v, fmacc)
            lacc = bias16
            for f in range(_NUM_FIELDS):
                lacc = lacc + linv[pl.ds(grp + f * 16, 16)]
            y = lacc + 0.5 * fmacc
            outb[pl.ds(t * 16, 16)] = 1.0 / (1.0 + jnp.exp(-y))
            return 0

        lax.fori_loop(0, _CHUNK // 16, group, 0)
        pltpu.sync_copy(outb, out_hbm.at[pl.ds(base + gc * _CHUNK, _CHUNK)])

    fire(0, 0)
    fire(1, 1)

    def step(g, _):
        for b in range(2):
            gc = 2 * g + b
            drain(b)
            compute(gc, b)

            @pl.when(gc + 2 < _NCHUNK)
            def _():
                fire(gc + 2, b)
        return 0

    lax.fori_loop(0, _NCHUNK // 2, step, 0)


def _fm_call(embed_table, lin_flat, idx_flat, bias16):
    mesh = plsc.VectorSubcoreMesh(core_axis_name="c", subcore_axis_name="s")
    kern = pl.kernel(
        _fm_body,
        mesh=mesh,
        out_type=jax.ShapeDtypeStruct((_B,), jnp.float32),
        compiler_params=pltpu.CompilerParams(use_tc_tiling_on_sc=False),
        scratch_types=[
            pltpu.VMEM((_RPC,), jnp.int32),
            pltpu.VMEM((_RPC,), jnp.int32),
            pltpu.VMEM((_RPC, _D), jnp.float32),
            pltpu.VMEM((_RPC, _D), jnp.float32),
            pltpu.VMEM((_RPC,), jnp.float32),
            pltpu.VMEM((_RPC,), jnp.float32),
            pltpu.VMEM((16,), jnp.float32),
            pltpu.VMEM((_CHUNK,), jnp.float32),
            pltpu.SemaphoreType.DMA,
            pltpu.SemaphoreType.DMA,
            pltpu.SemaphoreType.DMA,
            pltpu.SemaphoreType.DMA,
        ],
    )
    return kern(embed_table, lin_flat, idx_flat, bias16)


def kernel(xx, embed_table, linear_table, bias):
    offsets = (jnp.arange(_NUM_FIELDS, dtype=jnp.int32) * _FIELD_DIM)[None, :]
    idx = xx.astype(jnp.int32) + offsets
    idx_t = jnp.transpose(idx.reshape(_B // 16, 16, _NUM_FIELDS),
                          (0, 2, 1)).reshape(-1)
    lin_flat = linear_table.reshape(-1)
    bias16 = jnp.broadcast_to(bias.astype(jnp.float32), (16,))
    return _fm_call(embed_table, lin_flat, idx_t, bias16)
